# Initial kernel scaffold; baseline (speedup 1.0000x reference)
import jax
import jax.numpy as jnp
from jax.experimental import pallas as pl


def kernel(positions, node_attrs, edge_index, shifts, batch, ptr, atomic_energies, W_embed, W_up1, W_tp1, W_lin1, W_sc1, w_read1, W_up2, W_tp2, W_lin2, W_sc2, w_read2):
    raise NotImplementedError("write your pallas kernel here")


# speedup vs baseline: 1.0000x; 1.0000x over previous
import jax
import jax.numpy as jnp
import numpy as np
from jax.experimental import pallas as pl

N = 10000
E = 160000
D = 128
NB = 8
NSH = 4
G = 8
RMAX = 5.0
AVG_NEI = 16.0
SCALE = 0.804
SHIFT = -0.164


@jax.custom_vjp
def _scale_shift_pallas(x):
    n = x.shape[0]
    npad = ((n + 1023) // 1024) * 1024
    xp = jnp.pad(x, (0, npad - n)).reshape(npad // 128, 128)

    def body(x_ref, o_ref):
        o_ref[...] = SCALE * x_ref[...] + SHIFT

    yp = pl.pallas_call(
        body,
        out_shape=jax.ShapeDtypeStruct(xp.shape, xp.dtype),
    )(xp)
    return yp.reshape(-1)[:n]


def _ss_fwd(x):
    return _scale_shift_pallas(x), None


def _ss_bwd(_, g):
    return (SCALE * g,)


_scale_shift_pallas.defvjp(_ss_fwd, _ss_bwd)


def _bessel(r):
    n = jnp.arange(1, NB + 1, dtype=jnp.float32)
    return jnp.sqrt(2.0 / RMAX) * jnp.sin(n[None, :] * jnp.pi * r[:, None] / RMAX) / (r[:, None] + 1e-9)


def kernel(positions, node_attrs, edge_index, shifts, batch, ptr,
           atomic_energies, W_embed,
           W_up1, W_tp1, W_lin1, W_sc1, w_read1,
           W_up2, W_tp2, W_lin2, W_sc2, w_read2):
    num_graphs = int(ptr.shape[0]) - 1
    sender = edge_index[0]
    receiver = edge_index[1]
    layers = [(W_up1, W_tp1, W_lin1, W_sc1, w_read1),
              (W_up2, W_tp2, W_lin2, W_sc2, w_read2)]

    node_e0 = node_attrs @ atomic_energies
    e0 = jax.ops.segment_sum(node_e0, batch, num_segments=num_graphs)

    def interaction_energy(pos):
        vectors = pos[receiver] - pos[sender] + shifts
        lengths = jnp.sqrt(jnp.sum(vectors * vectors, axis=-1) + 1e-12)
        unit = vectors / lengths[:, None]
        edge_attrs = jnp.concatenate([jnp.ones((unit.shape[0], 1), dtype=unit.dtype), unit], axis=1)
        edge_feats = _bessel(lengths)
        node_feats = node_attrs @ W_embed
        pair_node_energy = jnp.zeros_like(node_e0)
        node_es_list = [pair_node_energy]
        node_feats_list = []
        for (W_up, W_tp, W_lin, W_sc, w_read) in layers:
            h = node_feats @ W_up
            tpw = edge_feats @ W_tp
            he = h[sender] * tpw
            msgs = [jax.ops.segment_sum(he * edge_attrs[:, k:k + 1], receiver, num_segments=pos.shape[0]) for k in range(NSH)]
            message = (jnp.concatenate(msgs, axis=-1) @ W_lin) / AVG_NEI
            sc = (node_feats @ W_sc) * jnp.sum(node_attrs, axis=-1, keepdims=True)
            node_feats = jnp.tanh(message) + sc
            node_feats_list.append(node_feats)
            node_es_list.append(node_feats @ w_read)
        node_feats_out = jnp.concatenate(node_feats_list, axis=-1)
        node_inter_es = jnp.sum(jnp.stack(node_es_list, axis=0), axis=0)
        node_inter_es = _scale_shift_pallas(node_inter_es)
        inter_e = jax.ops.segment_sum(node_inter_es, batch, num_segments=num_graphs)
        return jnp.sum(inter_e), (inter_e, node_inter_es, node_feats_out)

    (_, (inter_e, node_inter_es, node_feats_out)), grad_pos = jax.value_and_grad(interaction_energy, has_aux=True)(positions)
    forces = -grad_pos
    total_energy = e0 + inter_e
    node_energy = node_e0 + node_inter_es
    return (total_energy, node_energy, inter_e, forces, node_feats_out)


# speedup vs baseline: 2.1557x; 2.1556x over previous
import functools

import jax
import jax.numpy as jnp
import numpy as np
from jax import lax
from jax.experimental import pallas as pl
from jax.experimental.pallas import tpu as pltpu
from jax.experimental.pallas import tpu_sc as plsc

N = 10000
E = 160000
D = 128
NB = 8
NSH = 4
G = 8
RMAX = 5.0
AVG_NEI = 16.0
SCALE = 0.804
SHIFT = -0.164

NP = 10240
C_BESSEL = float(np.sqrt(2.0 / RMAX))
KJ = (np.arange(1, NB + 1) * np.pi / RMAX).astype(np.float32)

CHUNK = 128
NW = 32
BN = 1024
BE = 2000

_f32 = jnp.float32


def _sc_gather(table, idx, width):
    n = idx.shape[0]
    nchunks = n // CHUNK
    per_w = (nchunks + NW - 1) // NW
    mesh = plsc.VectorSubcoreMesh(core_axis_name="c", subcore_axis_name="s")

    @functools.partial(
        pl.kernel, mesh=mesh,
        out_type=jax.ShapeDtypeStruct((n, width), _f32),
        scratch_types=[
            pltpu.VMEM((CHUNK,), jnp.int32),
            pltpu.VMEM((CHUNK, width), _f32),
            pltpu.SemaphoreType.DMA,
        ],
        compiler_params=pltpu.CompilerParams(use_tc_tiling_on_sc=False),
    )
    def gk(table_hbm, idx_hbm, out_hbm, idx_v, rows_v, sem):
        w = lax.axis_index("s") * 2 + lax.axis_index("c")

        def body(i, carry):
            ch = w + NW * i

            @pl.when(ch < nchunks)
            def _():
                base = pl.multiple_of(ch * CHUNK, CHUNK)
                pltpu.sync_copy(idx_hbm.at[pl.ds(base, CHUNK)], idx_v)
                pltpu.async_copy(table_hbm.at[idx_v], rows_v, sem).wait()
                pltpu.sync_copy(rows_v, out_hbm.at[pl.ds(base, CHUNK)])

            return carry

        lax.fori_loop(0, per_w, body, None)

    return gk(table, idx)


def _sc_scatter_add(rows, idx, width, zblock):
    n = rows.shape[0]
    nchunks = n // CHUNK
    half = nchunks // 2
    per_s = (half + 15) // 16
    rps = NP // 16
    mesh = plsc.VectorSubcoreMesh(core_axis_name="c", subcore_axis_name="s")

    @functools.partial(
        pl.kernel, mesh=mesh,
        out_type=jax.ShapeDtypeStruct((2, NP, width), _f32),
        scratch_types=[
            pltpu.VMEM((CHUNK,), jnp.int32),
            pltpu.VMEM((CHUNK, width), _f32),
            pltpu.VMEM_SHARED((NP, width), _f32),
        ],
        compiler_params=pltpu.CompilerParams(use_tc_tiling_on_sc=False),
    )
    def sk(rows_hbm, idx_hbm, z_hbm, out_hbm, idx_v, rows_v, acc):
        c = lax.axis_index("c")
        s = lax.axis_index("s")
        row0 = pl.multiple_of(s * rps, rps)
        pltpu.sync_copy(z_hbm, acc.at[pl.ds(row0, rps)])
        plsc.subcore_barrier()

        def body(i, carry):
            j = s + 16 * i

            @pl.when(j < half)
            def _():
                base = pl.multiple_of((c * half + j) * CHUNK, CHUNK)
                pltpu.sync_copy(idx_hbm.at[pl.ds(base, CHUNK)], idx_v)
                pltpu.sync_copy(rows_hbm.at[pl.ds(base, CHUNK)], rows_v)
                pltpu.sync_copy(rows_v, acc.at[idx_v], add=True)

            return carry

        lax.fori_loop(0, per_s, body, None)
        plsc.subcore_barrier()
        pltpu.sync_copy(acc.at[pl.ds(row0, rps)], out_hbm.at[c, pl.ds(row0, rps)])

    return sk(rows, idx, zblock)


def _full(shape):
    return pl.BlockSpec(shape, lambda i: tuple(0 for _ in shape))


def _rows(b, *rest):
    shape = (b,) + rest
    return pl.BlockSpec(shape, lambda i: (i,) + tuple(0 for _ in rest))


def _call(body, grid, in_specs, out_specs, out_shape, *args):
    return pl.pallas_call(
        body, grid=(grid,), in_specs=in_specs, out_specs=out_specs,
        out_shape=out_shape)(*args)


def _k_node_prep(na, w_embed, ae2):
    def body(na_ref, we_ref, ae_ref, x0_ref, sq_ref):
        na_b = na_ref[...]
        x0_ref[...] = jnp.dot(na_b, we_ref[...], preferred_element_type=_f32,
                  precision=lax.Precision.HIGHEST)
        q = jnp.sum(na_b, axis=1, keepdims=True)
        e0 = jnp.sum(na_b * ae_ref[...], axis=1, keepdims=True)
        sq_ref[...] = jnp.concatenate(
            [q, e0, jnp.zeros((q.shape[0], 6), _f32)], axis=1)

    return _call(
        body, NP // BN,
        [_rows(BN, 4), _full((4, D)), _full((1, 4))],
        [_rows(BN, D), _rows(BN, 8)],
        [jax.ShapeDtypeStruct((NP, D), _f32), jax.ShapeDtypeStruct((NP, 8), _f32)],
        na, w_embed, ae2)


def _k_node_up(x, sq, w_up, w_sc):
    def body(x_ref, sq_ref, wu_ref, ws_ref, h_ref, sc_ref):
        x_b = x_ref[...]
        h_ref[...] = jnp.dot(x_b, wu_ref[...], preferred_element_type=_f32,
                  precision=lax.Precision.HIGHEST)
        sc_ref[...] = jnp.dot(x_b, ws_ref[...], preferred_element_type=_f32,
                  precision=lax.Precision.HIGHEST) \
            * sq_ref[...][:, 0:1]

    return _call(
        body, NP // BN,
        [_rows(BN, D), _rows(BN, 8), _full((D, D)), _full((D, D))],
        [_rows(BN, D), _rows(BN, D)],
        [jax.ShapeDtypeStruct((NP, D), _f32), jax.ShapeDtypeStruct((NP, D), _f32)],
        x, sq, w_up, w_sc)


def _k_geom(pr, ps, sh8):
    def body(pr_ref, ps_ref, sh_ref, a8_ref, f_ref):
        kj = (lax.broadcasted_iota(jnp.int32, (1, NB), 1) + 1).astype(_f32) * float(np.pi / RMAX)
        v = pr_ref[...][:, :3] - ps_ref[...][:, :3] + sh_ref[...][:, :3]
        l2 = jnp.sum(v * v, axis=1, keepdims=True) + 1e-12
        l = jnp.sqrt(l2)
        u = v / l
        a8_ref[...] = jnp.concatenate(
            [jnp.ones_like(l), u, l, jnp.zeros((v.shape[0], 3), _f32)], axis=1)
        f_ref[...] = C_BESSEL * jnp.sin(kj * l) / (l + 1e-9)

    return _call(
        body, E // BE,
        [_rows(BE, 16), _rows(BE, 16), _rows(BE, 8)],
        [_rows(BE, 8), _rows(BE, 8)],
        [jax.ShapeDtypeStruct((E, 8), _f32), jax.ShapeDtypeStruct((E, 8), _f32)],
        pr, ps, sh8)


def _k_edge_fwd(hs, f, a8, w_tp):
    def body(hs_ref, f_ref, a8_ref, wt_ref, he_ref, h1_ref, h2_ref, h3_ref):
        tpw = jnp.dot(f_ref[...], wt_ref[...], preferred_element_type=_f32,
                  precision=lax.Precision.HIGHEST)
        he = hs_ref[...] * tpw
        a = a8_ref[...]
        he_ref[...] = he
        h1_ref[...] = he * a[:, 1:2]
        h2_ref[...] = he * a[:, 2:3]
        h3_ref[...] = he * a[:, 3:4]

    shp = jax.ShapeDtypeStruct((E, D), _f32)
    return _call(
        body, E // BE,
        [_rows(BE, D), _rows(BE, 8), _rows(BE, 8), _full((NB, D))],
        [_rows(BE, D)] * 4, [shp, shp, shp, shp], hs, f, a8, w_tp)


def _k_node_update(p0, p1, p2, p3, sc, w_lin, w_read):
    def body(p0_ref, p1_ref, p2_ref, p3_ref, sc_ref, wl_ref, wr_ref,
             y_ref, t_ref, es_ref):
        wl = wl_ref[...]
        acc = jnp.zeros((p0_ref.shape[1], D), _f32)
        for k, ref in enumerate((p0_ref, p1_ref, p2_ref, p3_ref)):
            m = ref[0] + ref[1]
            acc = acc + jnp.dot(m, wl[k * D:(k + 1) * D, :],
                                preferred_element_type=_f32,
                  precision=lax.Precision.HIGHEST)
        t = jnp.tanh(acc * (1.0 / AVG_NEI))
        y = t + sc_ref[...]
        t_ref[...] = t
        y_ref[...] = y
        es = jnp.sum(y * wr_ref[...], axis=1, keepdims=True)
        es_ref[...] = jnp.concatenate(
            [es, jnp.zeros((es.shape[0], 7), _f32)], axis=1)

    pspec = pl.BlockSpec((2, BN, D), lambda i: (0, i, 0))
    return _call(
        body, NP // BN,
        [pspec, pspec, pspec, pspec, _rows(BN, D), _full((NSH * D, D)),
         _full((1, D))],
        [_rows(BN, D), _rows(BN, D), _rows(BN, 8)],
        [jax.ShapeDtypeStruct((NP, D), _f32), jax.ShapeDtypeStruct((NP, D), _f32),
         jax.ShapeDtypeStruct((NP, 8), _f32)],
        p0, p1, p2, p3, sc, w_lin, w_read)


def _k_bwd_node2(t2, w_lin2t, w_read2):
    def body(t_ref, wl_ref, wr_ref, mb_ref):
        t = t_ref[...]
        gmat = (1.0 - t * t) * (SCALE * wr_ref[...])
        mb_ref[...] = jnp.dot(gmat, wl_ref[...],
                              preferred_element_type=_f32,
                  precision=lax.Precision.HIGHEST) * (1.0 / AVG_NEI)

    return _call(
        body, NP // BN,
        [_rows(BN, D), _full((D, NSH * D)), _full((1, D))],
        [_rows(BN, NSH * D)],
        [jax.ShapeDtypeStruct((NP, NSH * D), _f32)],
        t2, w_lin2t, w_read2)[0]


def _k_bwd_node1(hp, sq, t1, w_up2t, w_sc2t, w_read1, w_read2, w_lin1t):
    def body(hp_ref, sq_ref, t_ref, wu_ref, ws_ref, wr1_ref, wr2_ref, wl_ref,
             mb_ref):
        hbar2 = hp_ref[0] + hp_ref[1]
        cv = SCALE * jnp.dot(wr2_ref[...], ws_ref[...],
                             preferred_element_type=_f32,
                  precision=lax.Precision.HIGHEST)
        q = sq_ref[...][:, 0:1]
        ybar1 = SCALE * wr1_ref[...] \
            + jnp.dot(hbar2, wu_ref[...], preferred_element_type=_f32,
                  precision=lax.Precision.HIGHEST) \
            + q * cv
        t = t_ref[...]
        gmat = ybar1 * (1.0 - t * t)
        mb_ref[...] = jnp.dot(gmat, wl_ref[...],
                              preferred_element_type=_f32,
                  precision=lax.Precision.HIGHEST) * (1.0 / AVG_NEI)

    return _call(
        body, NP // BN,
        [pl.BlockSpec((2, BN, D), lambda i: (0, i, 0)), _rows(BN, 8),
         _rows(BN, D), _full((D, D)), _full((D, D)), _full((1, D)),
         _full((1, D)), _full((D, NSH * D))],
        [_rows(BN, NSH * D)],
        [jax.ShapeDtypeStruct((NP, NSH * D), _f32)],
        hp, sq, t1, w_up2t, w_sc2t, w_read1, w_read2, w_lin1t)[0]


def _k_bwd_edge(mbr, he, hs, f, a8, w_tp, w_tpt, need_chw):
    def body(mbr_ref, he_ref, hs_ref, f_ref, a8_ref, wt_ref, wtt_ref, *outs):
        mbr_b = mbr_ref[...]
        he_b = he_ref[...]
        a = a8_ref[...]
        mb0 = mbr_b[:, 0:D]
        mb1 = mbr_b[:, D:2 * D]
        mb2 = mbr_b[:, 2 * D:3 * D]
        mb3 = mbr_b[:, 3 * D:4 * D]
        c = mb0 + a[:, 1:2] * mb1 + a[:, 2:3] * mb2 + a[:, 3:4] * mb3
        u1 = jnp.sum(mb1 * he_b, axis=1, keepdims=True)
        u2 = jnp.sum(mb2 * he_b, axis=1, keepdims=True)
        u3 = jnp.sum(mb3 * he_b, axis=1, keepdims=True)
        nb = u1.shape[0]
        outs[0][...] = jnp.concatenate(
            [u1, u2, u3, jnp.zeros((nb, 5), _f32)], axis=1)
        outs[1][...] = jnp.dot(c * hs_ref[...], wtt_ref[...],
                               preferred_element_type=_f32,
                  precision=lax.Precision.HIGHEST)
        if need_chw:
            tpw = jnp.dot(f_ref[...], wt_ref[...], preferred_element_type=_f32,
                  precision=lax.Precision.HIGHEST)
            outs[2][...] = c * tpw

    out_specs = [_rows(BE, 8), _rows(BE, 8)]
    out_shape = [jax.ShapeDtypeStruct((E, 8), _f32),
                 jax.ShapeDtypeStruct((E, 8), _f32)]
    if need_chw:
        out_specs.append(_rows(BE, D))
        out_shape.append(jax.ShapeDtypeStruct((E, D), _f32))
    return _call(
        body, E // BE,
        [_rows(BE, NSH * D), _rows(BE, D), _rows(BE, D), _rows(BE, 8),
         _rows(BE, 8), _full((NB, D)), _full((D, NB))],
        out_specs, out_shape,
        mbr, he, hs, f, a8, w_tp, w_tpt)


def _k_geom_bwd(a8, ub1, ub2, fb1, fb2):
    def body(a8_ref, ub1_ref, ub2_ref, fb1_ref, fb2_ref, vb_ref):
        kj = (lax.broadcasted_iota(jnp.int32, (1, NB), 1) + 1).astype(_f32) * float(np.pi / RMAX)
        a = a8_ref[...]
        u = a[:, 1:4]
        l = a[:, 4:5]
        rinv = 1.0 / (l + 1e-9)
        fbar = fb1_ref[...] + fb2_ref[...]
        dfdl = C_BESSEL * (kj * jnp.cos(kj * l) * rinv
                           - jnp.sin(kj * l) * rinv * rinv)
        lbar = jnp.sum(fbar * dfdl, axis=1, keepdims=True)
        ub = ub1_ref[...][:, 0:3] + ub2_ref[...][:, 0:3]
        udot = jnp.sum(ub * u, axis=1, keepdims=True)
        vbar = (ub - udot * u) / l + lbar * u
        vb_ref[...] = jnp.concatenate(
            [vbar, jnp.zeros((vbar.shape[0], 13), _f32)], axis=1)

    return _call(
        body, E // BE,
        [_rows(BE, 8), _rows(BE, 8), _rows(BE, 8), _rows(BE, 8), _rows(BE, 8)],
        [_rows(BE, 16)],
        [jax.ShapeDtypeStruct((E, 16), _f32)],
        a8, ub1, ub2, fb1, fb2)[0]


def _k_forces(pr, ps):
    def body(pr_ref, ps_ref, f_ref):
        f_ref[...] = -(pr_ref[0] + pr_ref[1] - ps_ref[0] - ps_ref[1])

    pspec = pl.BlockSpec((2, BN, 16), lambda i: (0, i, 0))
    return _call(
        body, NP // BN, [pspec, pspec], [_rows(BN, 16)],
        [jax.ShapeDtypeStruct((NP, 16), _f32)],
        pr, ps)[0]


def _k_energy(es1, es2, sq, batch2):
    def body(es1_ref, es2_ref, sq_ref, b_ref, ne_ref, sums_ref):
        nin = SCALE * (es1_ref[...][:, 0:1] + es2_ref[...][:, 0:1]) + SHIFT
        e0n = sq_ref[...][:, 1:2]
        nb = nin.shape[0]
        ne_ref[...] = jnp.concatenate(
            [e0n + nin, jnp.zeros((nb, 7), _f32)], axis=1)
        giota = lax.broadcasted_iota(jnp.int32, (nb, G), 1)
        onehot = (b_ref[...] == giota).astype(_f32)
        ig = jnp.dot(nin.T, onehot, preferred_element_type=_f32,
                  precision=lax.Precision.HIGHEST)[0]
        eg = jnp.dot(e0n.T, onehot, preferred_element_type=_f32,
                  precision=lax.Precision.HIGHEST)[0]
        blk = jnp.stack([ig, eg, ig + eg], axis=1)
        blk = jnp.concatenate([blk, jnp.zeros((G, 5), _f32)], axis=1)

        @pl.when(pl.program_id(0) == 0)
        def _():
            sums_ref[...] = jnp.zeros((G, 8), _f32)

        sums_ref[...] += blk

    return _call(
        body, NP // BN,
        [_rows(BN, 8), _rows(BN, 8), _rows(BN, 8),
         pl.BlockSpec((BN, 1), lambda i: (i, 0))],
        [_rows(BN, 8), pl.BlockSpec((G, 8), lambda i: (0, 0))],
        [jax.ShapeDtypeStruct((NP, 8), _f32), jax.ShapeDtypeStruct((G, 8), _f32)],
        es1, es2, sq, batch2)


def kernel(positions, node_attrs, edge_index, shifts, batch, ptr,
           atomic_energies, W_embed,
           W_up1, W_tp1, W_lin1, W_sc1, w_read1,
           W_up2, W_tp2, W_lin2, W_sc2, w_read2):
    sender = edge_index[0].astype(jnp.int32)
    receiver = edge_index[1].astype(jnp.int32)

    pos16 = jnp.zeros((NP, 16), _f32).at[:N, :3].set(positions)
    na_p = jnp.zeros((NP, 4), _f32).at[:N].set(node_attrs)
    sh8 = jnp.zeros((E, 8), _f32).at[:, :3].set(shifts)
    batch2 = jnp.full((NP, 1), G, jnp.int32).at[:N, 0].set(batch.astype(jnp.int32))
    ae2 = atomic_energies.reshape(1, 4)
    wr1 = w_read1.reshape(1, D)
    wr2 = w_read2.reshape(1, D)
    z128 = jnp.zeros((NP // 16, D), _f32)
    z16 = jnp.zeros((NP // 16, 16), _f32)

    pr = _sc_gather(pos16, receiver, 16)
    ps = _sc_gather(pos16, sender, 16)
    a8, f = _k_geom(pr, ps, sh8)

    x0, sq = _k_node_prep(na_p, W_embed, ae2)

    def layer_fwd(x, w_up, w_tp, w_lin, w_sc, wr):
        h, sc = _k_node_up(x, sq, w_up, w_sc)
        hs = _sc_gather(h, sender, D)
        he, h1, h2, h3 = _k_edge_fwd(hs, f, a8, w_tp)
        p0 = _sc_scatter_add(he, receiver, D, z128)
        p1 = _sc_scatter_add(h1, receiver, D, z128)
        p2 = _sc_scatter_add(h2, receiver, D, z128)
        p3 = _sc_scatter_add(h3, receiver, D, z128)
        y, t, es = _k_node_update(p0, p1, p2, p3, sc, w_lin, wr)
        return y, t, es, he, hs

    y1, t1, es1, he1, hs1 = layer_fwd(x0, W_up1, W_tp1, W_lin1, W_sc1, wr1)
    y2, t2, es2, he2, hs2 = layer_fwd(y1, W_up2, W_tp2, W_lin2, W_sc2, wr2)

    ne8, sums = _k_energy(es1, es2, sq, batch2)

    wl1t = W_lin1.T
    wl2t = W_lin2.T
    wu2t = W_up2.T
    ws2t = W_sc2.T
    wt1t = W_tp1.T
    wt2t = W_tp2.T
    mb2 = _k_bwd_node2(t2, wl2t, wr2)
    mbr2 = _sc_gather(mb2, receiver, NSH * D)
    ub2, fb2, chw = _k_bwd_edge(mbr2, he2, hs2, f, a8, W_tp2, wt2t, True)
    hp = _sc_scatter_add(chw, sender, D, z128)
    mb1 = _k_bwd_node1(hp, sq, t1, wu2t, ws2t, wr1, wr2, wl1t)
    mbr1 = _sc_gather(mb1, receiver, NSH * D)
    ub1, fb1 = _k_bwd_edge(mbr1, he1, hs1, f, a8, W_tp1, wt1t, False)

    vb16 = _k_geom_bwd(a8, ub1, ub2, fb1, fb2)
    pr_acc = _sc_scatter_add(vb16, receiver, 16, z16)
    ps_acc = _sc_scatter_add(vb16, sender, 16, z16)
    f16 = _k_forces(pr_acc, ps_acc)

    total_energy = sums[:, 2]
    inter_e = sums[:, 0]
    node_energy = ne8[:N, 0]
    forces = f16[:N, :3]
    node_feats_out = jnp.concatenate([y1[:N], y2[:N]], axis=1)
    return (total_energy, node_energy, inter_e, forces, node_feats_out)
